# baseline (speedup 1.0000x reference)
import numpy as np
import jax
import jax.numpy as jnp
from jax.experimental import pallas as pl
from jax.experimental.pallas import tpu as pltpu

_B = 512
_NE = 18
_NN = 4
_NP = _NE * _NE
_NPN = _NN * _NE
_EB = 128
_NRBF = 20
_NF = _NRBF + 1
_NFP = 24
_CUT = 5.0
_L = 3
_NBF = 8
_W = 8
_F32 = jnp.float32


def _silu(x):
    return x * jax.nn.sigmoid(x)


def _mm(a, b):
    return jnp.dot(a, b, preferred_element_type=jnp.float32)


def _mmT(a, b):
    return jax.lax.dot_general(a, b, (((0,), (0,)), ((), ())),
                               preferred_element_type=jnp.float32)


def _update(s, vx, vy, vz, U, V, w1, b1, w2, b2):
    n = _W * _NE
    vcat = jnp.concatenate([vx, vy, vz], axis=0)
    Uv = _mm(vcat, U)
    Vv = _mm(vcat, V)
    Uvx, Uvy, Uvz = Uv[:n], Uv[n:2 * n], Uv[2 * n:]
    Vvx, Vvy, Vvz = Vv[:n], Vv[n:2 * n], Vv[2 * n:]
    Vn = jnp.sqrt(Vvx * Vvx + Vvy * Vvy + Vvz * Vvz + 1e-8)
    cat = jnp.concatenate([s, Vn], axis=-1)
    a = _mm(_silu(_mm(cat, w1) + b1), w2) + b2
    uvdot = Uvx * Vvx + Uvy * Vvy + Uvz * Vvz
    s = s + a[:, :_EB] + a[:, _EB:2 * _EB] * uvdot
    g = a[:, 2 * _EB:]
    return s, vx + g * Uvx, vy + g * Uvy, vz + g * Uvz


def _features(dx, dy, dz, mask):
    d = jnp.sqrt(dx * dx + dy * dy + dz * dz + 1e-12)
    dc = jnp.maximum(d, 1e-6)
    inv = 1.0 / dc
    th = d * (np.pi / _CUT)
    s1 = jnp.sin(th)
    c1 = jnp.cos(th)
    fc = jnp.where(d < _CUT, 0.5 * (c1 + 1.0), 0.0)
    if mask is not None:
        fc = fc * mask
    g = inv * fc
    base = []
    sq, cq = s1, c1
    base.append(sq * g)
    for _ in range(_NRBF - 1):
        sq, cq = sq * c1 + cq * s1, cq * c1 - sq * s1
        base.append(sq * g)
    base.append(fc)
    ux = dx * inv
    uy = dy * inv
    uz = dz * inv
    pad = [jnp.zeros_like(fc)] * (_NFP - _NF)
    return (base + pad + [r * ux for r in base] + pad
            + [r * uy for r in base] + pad
            + [r * uz for r in base] + pad)


def _painn_kernel(
    rsl_ref, rnt_ref, X_ref, Y_ref, TE_ref, TJ_ref, TN_ref, EN_ref, mask_ref,
    ee_w1, ee_b1, ee_w2, ee_b2, ee_wb,
    ne_w1, ne_b1, ne_w2, ne_b2, ne_wb,
    ue_U, ue_V, ue_w1, ue_b1, ue_w2, ue_b2,
    un_U, un_V, un_w1, un_b1, un_w2, un_b2,
    jw1, jb1, jw2, jb2, bfw,
    jas_ref, bx_ref, by_ref, bz_ref,
    fa_ref, fan_ref, phi_ref, pv_ref,
    ds_ref, dvx_ref, dvy_ref, dvz_ref,
):
    W = _W
    n = W * _NE

    rx = rsl_ref[:, 0, :]
    ry = rsl_ref[:, 1, :]
    rz = rsl_ref[:, 2, :]
    TE = TE_ref[:]
    TJ = TJ_ref[:]
    dx = _mm(rx, TE) - _mm(rx, TJ)
    dy = _mm(ry, TE) - _mm(ry, TJ)
    dz = _mm(rz, TE) - _mm(rz, TJ)
    rows = _features(dx, dy, dz, mask_ref[:])
    for q in range(4 * _NFP):
        fa_ref[:, q, :] = rows[q]

    TN = TN_ref[:]
    EN = EN_ref[:]
    rnt = rnt_ref[:]
    dxn = _mm(rx, TN) - _mm(rnt[0:1, :], EN)
    dyn = _mm(ry, TN) - _mm(rnt[1:2, :], EN)
    dzn = _mm(rz, TN) - _mm(rnt[2:3, :], EN)
    rowsn = _features(dxn, dyn, dzn, None)
    for q in range(4 * _NFP):
        fan_ref[:, q, :] = rowsn[q]

    Yv = Y_ref[:]

    s = jnp.broadcast_to(X_ref[:], (n, _EB))
    vx = jnp.zeros((n, _EB), _F32)
    vy = jnp.zeros((n, _EB), _F32)
    vz = jnp.zeros((n, _EB), _F32)

    for l in range(_L):
        h = _silu(_mm(s, ee_w1[l]) + ee_b1[l])
        phi = _mm(h, ee_w2[l]) + ee_b2[l]
        phi_ref[:] = phi.reshape(W, _NE, 3 * _EB)
        pvv = phi[:, _EB:2 * _EB]
        pv_ref[:] = jnp.concatenate(
            [pvv * vx, pvv * vy, pvv * vz], axis=-1).reshape(W, _NE, 3 * _EB)
        wbig = ee_wb[l]

        def ee_body(b, carry):
            A = fa_ref[b]
            o3 = _mmT(A, wbig).reshape(_NE, _NE, 5 * _EB)
            phib = phi_ref[b]
            pvb = pv_ref[b]
            pd = phib[None, :, 2 * _EB:]
            wvv = o3[:, :, _EB:2 * _EB]
            xs = phib[None, :, :_EB] * o3[:, :, :_EB]
            tx = pvb[None, :, :_EB] * wvv + pd * o3[:, :, 2 * _EB:3 * _EB]
            ty = (pvb[None, :, _EB:2 * _EB] * wvv
                  + pd * o3[:, :, 3 * _EB:4 * _EB])
            tz = (pvb[None, :, 2 * _EB:] * wvv
                  + pd * o3[:, :, 4 * _EB:])
            cat = jnp.concatenate([xs, tx, ty, tz], axis=-1)
            r = _mm(TE, cat.reshape(_NP, 4 * _EB))
            ds_ref[b] = r[:, :_EB]
            dvx_ref[b] = r[:, _EB:2 * _EB]
            dvy_ref[b] = r[:, 2 * _EB:3 * _EB]
            dvz_ref[b] = r[:, 3 * _EB:]
            return carry

        jax.lax.fori_loop(0, W, ee_body, 0)
        s = s + ds_ref[:].reshape(n, _EB)
        vx = vx + dvx_ref[:].reshape(n, _EB)
        vy = vy + dvy_ref[:].reshape(n, _EB)
        vz = vz + dvz_ref[:].reshape(n, _EB)

        s, vx, vy, vz = _update(s, vx, vy, vz, ue_U[l], ue_V[l], ue_w1[l],
                                ue_b1[l], ue_w2[l], ue_b2[l])

        hn = _silu(_mm(Yv, ne_w1[l]) + ne_b1[l])
        phin = _mm(hn, ne_w2[l]) + ne_b2[l]
        wbign = ne_wb[l]
        pdn = phin[:, None, 2 * _EB:]
        psn = phin[:, None, :_EB]

        def ne_body(b, carry):
            A = fan_ref[b]
            o3 = _mmT(A, wbign).reshape(_NN, _NE, 4 * _EB)
            xs = psn * o3[:, :, :_EB]
            tx = pdn * o3[:, :, _EB:2 * _EB]
            ty = pdn * o3[:, :, 2 * _EB:3 * _EB]
            tz = pdn * o3[:, :, 3 * _EB:]
            cat = jnp.concatenate([xs, tx, ty, tz], axis=-1)
            r = _mm(TN, cat.reshape(_NPN, 4 * _EB))
            ds_ref[b] = r[:, :_EB]
            dvx_ref[b] = r[:, _EB:2 * _EB]
            dvy_ref[b] = r[:, 2 * _EB:3 * _EB]
            dvz_ref[b] = r[:, 3 * _EB:]
            return carry

        jax.lax.fori_loop(0, W, ne_body, 0)
        s = s + ds_ref[:].reshape(n, _EB)
        vx = vx + dvx_ref[:].reshape(n, _EB)
        vy = vy + dvy_ref[:].reshape(n, _EB)
        vz = vz + dvz_ref[:].reshape(n, _EB)

        s, vx, vy, vz = _update(s, vx, vy, vz, un_U[l], un_V[l], un_w1[l],
                                un_b1[l], un_w2[l], un_b2[l])

    hsum = jnp.sum(s.reshape(W, _NE, _EB), axis=1)
    jas_ref[:] = _mm(_silu(_mm(hsum, jw1[:]) + jb1[:]), jw2[:]) + jb2[:]
    bx_ref[:] = _mm(vx, bfw[:]).reshape(W, _NE, _NBF)
    by_ref[:] = _mm(vy, bfw[:]).reshape(W, _NE, _NBF)
    bz_ref[:] = _mm(vz, bfw[:]).reshape(W, _NE, _NBF)


@jax.jit
def kernel(rs, rn, params):
    p = params
    L = _L
    ee, ne = p['msg_ee'], p['msg_ne']
    ue, un = p['upd_e'], p['upd_n']

    def blockw(q, n_dir_lanes_prefix):
        wfa = jnp.concatenate([q['wf'], q['bf'][:, None, :]], axis=1)
        return wfa

    def padrows(m):
        return jnp.concatenate(
            [m, jnp.zeros((L, _NFP - _NF) + m.shape[2:], _F32)], axis=1)

    ee_wfa = blockw(ee, 0)
    ne_wfa = blockw(ne, 0)
    z21 = jnp.zeros((L, _NF, _EB), _F32)
    g0 = jnp.concatenate([ee_wfa[:, :, :2 * _EB], z21, z21, z21], axis=2)
    vd = ee_wfa[:, :, 2 * _EB:]
    z2 = jnp.zeros((L, _NF, 2 * _EB), _F32)
    gx = jnp.concatenate([z2, vd, z21, z21], axis=2)
    gy = jnp.concatenate([z2, z21, vd, z21], axis=2)
    gz = jnp.concatenate([z2, z21, z21, vd], axis=2)
    ee_wb = jnp.concatenate(
        [padrows(g0), padrows(gx), padrows(gy), padrows(gz)], axis=1)
    vdn = ne_wfa[:, :, 2 * _EB:]
    z1 = jnp.zeros((L, _NF, _EB), _F32)
    g0n = jnp.concatenate([ne_wfa[:, :, :_EB], z1, z1, z1], axis=2)
    gxn = jnp.concatenate([z1, vdn, z1, z1], axis=2)
    gyn = jnp.concatenate([z1, z1, vdn, z1], axis=2)
    gzn = jnp.concatenate([z1, z1, z1, vdn], axis=2)
    ne_wb = jnp.concatenate(
        [padrows(g0n), padrows(gxn), padrows(gyn), padrows(gzn)], axis=1)

    ii, jj = np.meshgrid(np.arange(_NE), np.arange(_NE), indexing='ij')
    TE = np.zeros((_NE, _NP), np.float32)
    TE[ii.reshape(-1), np.arange(_NP)] = 1.0
    TJ = np.zeros((_NE, _NP), np.float32)
    TJ[jj.reshape(-1), np.arange(_NP)] = 1.0
    mask = (ii != jj).reshape(1, _NP).astype(np.float32)
    aa, ei = np.meshgrid(np.arange(_NN), np.arange(_NE), indexing='ij')
    TN = np.zeros((_NE, _NPN), np.float32)
    TN[ei.reshape(-1), np.arange(_NPN)] = 1.0
    EN = np.zeros((_NN, _NPN), np.float32)
    EN[aa.reshape(-1), np.arange(_NPN)] = 1.0

    ops = [
        jnp.swapaxes(rs, 1, 2),
        rn.T,
        p['X'],
        p['Y'],
        jnp.asarray(TE), jnp.asarray(TJ), jnp.asarray(TN), jnp.asarray(EN),
        jnp.asarray(mask),
        ee['w1'], ee['b1'].reshape(L, 1, _EB),
        ee['w2'], ee['b2'].reshape(L, 1, 3 * _EB),
        ee_wb,
        ne['w1'], ne['b1'].reshape(L, 1, _EB),
        ne['w2'], ne['b2'].reshape(L, 1, 3 * _EB),
        ne_wb,
        ue['U'], ue['V'],
        ue['w1'], ue['b1'].reshape(L, 1, _EB),
        ue['w2'], ue['b2'].reshape(L, 1, 3 * _EB),
        un['U'], un['V'],
        un['w1'], un['b1'].reshape(L, 1, _EB),
        un['w2'], un['b2'].reshape(L, 1, 3 * _EB),
        p['jw1'], p['jb1'].reshape(1, _EB),
        p['jw2'], p['jb2'].reshape(1, 1),
        p['bf'],
    ]

    def rep(shape):
        nd = len(shape)
        return pl.BlockSpec(shape, lambda b, _nd=nd: (0,) * _nd)

    in_specs = [pl.BlockSpec((_W, 3, _NE), lambda b: (b, 0, 0))]
    in_specs += [rep(o.shape) for o in ops[1:]]

    scratch = [
        pltpu.VMEM((_W, 4 * _NFP, _NP), _F32),
        pltpu.VMEM((_W, 4 * _NFP, _NPN), _F32),
        pltpu.VMEM((_W, _NE, 3 * _EB), _F32),
        pltpu.VMEM((_W, _NE, 3 * _EB), _F32),
        pltpu.VMEM((_W, _NE, _EB), _F32),
        pltpu.VMEM((_W, _NE, _EB), _F32),
        pltpu.VMEM((_W, _NE, _EB), _F32),
        pltpu.VMEM((_W, _NE, _EB), _F32),
    ]

    bfspec = pl.BlockSpec((_W, _NE, _NBF), lambda b: (b, 0, 0))
    bfshape = jax.ShapeDtypeStruct((_B, _NE, _NBF), _F32)
    jas, bx, by, bz = pl.pallas_call(
        _painn_kernel,
        grid=(_B // _W,),
        in_specs=in_specs,
        out_specs=[pl.BlockSpec((_W, 1), lambda b: (b, 0)),
                   bfspec, bfspec, bfspec],
        out_shape=[jax.ShapeDtypeStruct((_B, 1), _F32),
                   bfshape, bfshape, bfshape],
        scratch_shapes=scratch,
        compiler_params=pltpu.CompilerParams(
            dimension_semantics=("parallel",),
        ),
    )(*ops)

    jastrow = jas[:, 0]
    backflow = jnp.stack([bx, by, bz], axis=-1).transpose(0, 2, 1, 3)
    return jastrow, backflow


# speedup vs baseline: 1.1909x; 1.1909x over previous
import numpy as np
import jax
import jax.numpy as jnp
from jax.experimental import pallas as pl
from jax.experimental.pallas import tpu as pltpu

_B = 512
_NE = 18
_NN = 4
_NP = _NE * _NE
_NPN = _NN * _NE
_EB = 128
_NRBF = 20
_NF = _NRBF + 1
_NFP = 24
_CUT = 5.0
_L = 3
_NBF = 8
_W = 8
_F32 = jnp.float32
_BF16 = jnp.bfloat16


def _silu(x):
    return x * jax.nn.sigmoid(x)


def _mm(a, b):
    return jnp.dot(a, b, preferred_element_type=jnp.float32)


def _mmT(a, b):
    return jax.lax.dot_general(a, b, (((0,), (0,)), ((), ())),
                               preferred_element_type=jnp.float32)


def _update(s, vx, vy, vz, U, V, w1, b1, w2, b2):
    n = _W * _NE
    vcat = jnp.concatenate([vx, vy, vz], axis=0)
    Uv = _mm(vcat, U)
    Vv = _mm(vcat, V)
    Uvx, Uvy, Uvz = Uv[:n], Uv[n:2 * n], Uv[2 * n:]
    Vvx, Vvy, Vvz = Vv[:n], Vv[n:2 * n], Vv[2 * n:]
    Vn = jnp.sqrt(Vvx * Vvx + Vvy * Vvy + Vvz * Vvz + 1e-8)
    cat = jnp.concatenate([s, Vn], axis=-1)
    a = _mm(_silu(_mm(cat, w1) + b1), w2) + b2
    uvdot = Uvx * Vvx + Uvy * Vvy + Uvz * Vvz
    s = s + a[:, :_EB] + a[:, _EB:2 * _EB] * uvdot
    g = a[:, 2 * _EB:]
    return s, vx + g * Uvx, vy + g * Uvy, vz + g * Uvz


def _features(dx, dy, dz, mask):
    d = jnp.sqrt(dx * dx + dy * dy + dz * dz + 1e-12)
    dc = jnp.maximum(d, 1e-6)
    inv = 1.0 / dc
    th = d * (np.pi / _CUT)
    s1 = jnp.sin(th)
    c1 = jnp.cos(th)
    fc = jnp.where(d < _CUT, 0.5 * (c1 + 1.0), 0.0)
    if mask is not None:
        fc = fc * mask
    g = inv * fc
    base = []
    sq, cq = s1, c1
    base.append(sq * g)
    for _ in range(_NRBF - 1):
        sq, cq = sq * c1 + cq * s1, cq * c1 - sq * s1
        base.append(sq * g)
    base.append(fc)
    ux = dx * inv
    uy = dy * inv
    uz = dz * inv
    pad = [jnp.zeros_like(fc)] * (_NFP - _NF)
    return (base + pad + [r * ux for r in base] + pad
            + [r * uy for r in base] + pad
            + [r * uz for r in base] + pad)


def _painn_kernel(
    rsl_ref, rnt_ref, X_ref, Y_ref, TE_ref, TJ_ref, TN_ref, EN_ref, mask_ref,
    ee_w1, ee_b1, ee_w2, ee_b2, ee_wfa,
    ne_w1, ne_b1, ne_w2, ne_b2, ne_wfa,
    ue_U, ue_V, ue_w1, ue_b1, ue_w2, ue_b2,
    un_U, un_V, un_w1, un_b1, un_w2, un_b2,
    jw1, jb1, jw2, jb2, bfw,
    jas_ref, bx_ref, by_ref, bz_ref,
    fa_ref, fan_ref, phi_ref, pv_ref,
    ds_ref, dvx_ref, dvy_ref, dvz_ref,
):
    W = _W
    n = W * _NE

    rx = rsl_ref[:, 0, :]
    ry = rsl_ref[:, 1, :]
    rz = rsl_ref[:, 2, :]
    TE = TE_ref[:]
    TJ = TJ_ref[:]
    dx = _mm(rx, TE) - _mm(rx, TJ)
    dy = _mm(ry, TE) - _mm(ry, TJ)
    dz = _mm(rz, TE) - _mm(rz, TJ)
    rows = _features(dx, dy, dz, mask_ref[:])
    for q in range(4 * _NFP):
        fa_ref[:, q, :] = rows[q].astype(_BF16)

    TN = TN_ref[:]
    EN = EN_ref[:]
    rnt = rnt_ref[:]
    dxn = _mm(rx, TN) - _mm(rnt[0:1, :], EN)
    dyn = _mm(ry, TN) - _mm(rnt[1:2, :], EN)
    dzn = _mm(rz, TN) - _mm(rnt[2:3, :], EN)
    rowsn = _features(dxn, dyn, dzn, None)
    for q in range(4 * _NFP):
        fan_ref[:, q, :] = rowsn[q].astype(_BF16)

    Yv = Y_ref[:]

    s = jnp.broadcast_to(X_ref[:], (n, _EB))
    vx = jnp.zeros((n, _EB), _F32)
    vy = jnp.zeros((n, _EB), _F32)
    vz = jnp.zeros((n, _EB), _F32)

    for l in range(_L):
        h = _silu(_mm(s, ee_w1[l]) + ee_b1[l])
        phi = _mm(h, ee_w2[l]) + ee_b2[l]
        phi_ref[:] = phi.reshape(W, _NE, 3 * _EB)
        pvv = phi[:, _EB:2 * _EB]
        pv_ref[:] = jnp.concatenate(
            [pvv * vx, pvv * vy, pvv * vz], axis=-1).reshape(W, _NE, 3 * _EB)
        wfa = ee_wfa[l]
        wfvd = wfa[:, 2 * _EB:]

        def ee_body(b, carry):
            A = fa_ref[b]
            wfb = _mmT(A[:_NFP], wfa).reshape(_NE, _NE, 3 * _EB)
            wdx = _mmT(A[_NFP:2 * _NFP], wfvd).reshape(_NE, _NE, _EB)
            wdy = _mmT(A[2 * _NFP:3 * _NFP], wfvd).reshape(_NE, _NE, _EB)
            wdz = _mmT(A[3 * _NFP:], wfvd).reshape(_NE, _NE, _EB)
            phib = phi_ref[b]
            pvb = pv_ref[b]
            pd = phib[None, :, 2 * _EB:]
            ds_ref[b] = jnp.sum(phib[None, :, :_EB] * wfb[:, :, :_EB],
                                axis=1)
            wvv = wfb[:, :, _EB:2 * _EB]
            dvx_ref[b] = jnp.sum(pvb[None, :, :_EB] * wvv + pd * wdx, axis=1)
            dvy_ref[b] = jnp.sum(pvb[None, :, _EB:2 * _EB] * wvv + pd * wdy,
                                 axis=1)
            dvz_ref[b] = jnp.sum(pvb[None, :, 2 * _EB:] * wvv + pd * wdz,
                                 axis=1)
            return carry

        jax.lax.fori_loop(0, W, ee_body, 0)
        s = s + ds_ref[:].reshape(n, _EB)
        vx = vx + dvx_ref[:].reshape(n, _EB)
        vy = vy + dvy_ref[:].reshape(n, _EB)
        vz = vz + dvz_ref[:].reshape(n, _EB)

        s, vx, vy, vz = _update(s, vx, vy, vz, ue_U[l], ue_V[l], ue_w1[l],
                                ue_b1[l], ue_w2[l], ue_b2[l])

        hn = _silu(_mm(Yv, ne_w1[l]) + ne_b1[l])
        phin = _mm(hn, ne_w2[l]) + ne_b2[l]
        wfna = ne_wfa[l]
        wfnvd = wfna[:, 2 * _EB:]

        def ne_body(b, carry):
            A = fan_ref[b]
            wfb = _mmT(A[:_NFP], wfna).reshape(_NN, _NE, 3 * _EB)
            wdx = _mmT(A[_NFP:2 * _NFP], wfnvd).reshape(_NN, _NE, _EB)
            wdy = _mmT(A[2 * _NFP:3 * _NFP], wfnvd).reshape(_NN, _NE, _EB)
            wdz = _mmT(A[3 * _NFP:], wfnvd).reshape(_NN, _NE, _EB)
            pd = phin[:, None, 2 * _EB:]
            ds_ref[b] = jnp.sum(phin[:, None, :_EB] * wfb[:, :, :_EB],
                                axis=0)
            dvx_ref[b] = jnp.sum(pd * wdx, axis=0)
            dvy_ref[b] = jnp.sum(pd * wdy, axis=0)
            dvz_ref[b] = jnp.sum(pd * wdz, axis=0)
            return carry

        jax.lax.fori_loop(0, W, ne_body, 0)
        s = s + ds_ref[:].reshape(n, _EB)
        vx = vx + dvx_ref[:].reshape(n, _EB)
        vy = vy + dvy_ref[:].reshape(n, _EB)
        vz = vz + dvz_ref[:].reshape(n, _EB)

        s, vx, vy, vz = _update(s, vx, vy, vz, un_U[l], un_V[l], un_w1[l],
                                un_b1[l], un_w2[l], un_b2[l])

    hsum = jnp.sum(s.reshape(W, _NE, _EB), axis=1)
    jas_ref[:] = _mm(_silu(_mm(hsum, jw1[:]) + jb1[:]), jw2[:]) + jb2[:]
    bx_ref[:] = _mm(vx, bfw[:]).reshape(W, _NE, _NBF)
    by_ref[:] = _mm(vy, bfw[:]).reshape(W, _NE, _NBF)
    bz_ref[:] = _mm(vz, bfw[:]).reshape(W, _NE, _NBF)


@jax.jit
def kernel(rs, rn, params):
    p = params
    L = _L
    ee, ne = p['msg_ee'], p['msg_ne']
    ue, un = p['upd_e'], p['upd_n']

    zpad = jnp.zeros((L, _NFP - _NF, 3 * _EB), _F32)
    ee_wfa = jnp.concatenate(
        [ee['wf'], ee['bf'][:, None, :], zpad], axis=1).astype(_BF16)
    ne_wfa = jnp.concatenate(
        [ne['wf'], ne['bf'][:, None, :], zpad], axis=1).astype(_BF16)

    ii, jj = np.meshgrid(np.arange(_NE), np.arange(_NE), indexing='ij')
    TE = np.zeros((_NE, _NP), np.float32)
    TE[ii.reshape(-1), np.arange(_NP)] = 1.0
    TJ = np.zeros((_NE, _NP), np.float32)
    TJ[jj.reshape(-1), np.arange(_NP)] = 1.0
    mask = (ii != jj).reshape(1, _NP).astype(np.float32)
    aa, ei = np.meshgrid(np.arange(_NN), np.arange(_NE), indexing='ij')
    TN = np.zeros((_NE, _NPN), np.float32)
    TN[ei.reshape(-1), np.arange(_NPN)] = 1.0
    EN = np.zeros((_NN, _NPN), np.float32)
    EN[aa.reshape(-1), np.arange(_NPN)] = 1.0

    ops = [
        jnp.swapaxes(rs, 1, 2),
        rn.T,
        p['X'],
        p['Y'],
        jnp.asarray(TE), jnp.asarray(TJ), jnp.asarray(TN), jnp.asarray(EN),
        jnp.asarray(mask),
        ee['w1'], ee['b1'].reshape(L, 1, _EB),
        ee['w2'], ee['b2'].reshape(L, 1, 3 * _EB),
        ee_wfa,
        ne['w1'], ne['b1'].reshape(L, 1, _EB),
        ne['w2'], ne['b2'].reshape(L, 1, 3 * _EB),
        ne_wfa,
        ue['U'], ue['V'],
        ue['w1'], ue['b1'].reshape(L, 1, _EB),
        ue['w2'], ue['b2'].reshape(L, 1, 3 * _EB),
        un['U'], un['V'],
        un['w1'], un['b1'].reshape(L, 1, _EB),
        un['w2'], un['b2'].reshape(L, 1, 3 * _EB),
        p['jw1'], p['jb1'].reshape(1, _EB),
        p['jw2'], p['jb2'].reshape(1, 1),
        p['bf'],
    ]

    def rep(shape):
        nd = len(shape)
        return pl.BlockSpec(shape, lambda b, _nd=nd: (0,) * _nd)

    in_specs = [pl.BlockSpec((_W, 3, _NE), lambda b: (b, 0, 0))]
    in_specs += [rep(o.shape) for o in ops[1:]]

    scratch = [
        pltpu.VMEM((_W, 4 * _NFP, _NP), _BF16),
        pltpu.VMEM((_W, 4 * _NFP, _NPN), _BF16),
        pltpu.VMEM((_W, _NE, 3 * _EB), _F32),
        pltpu.VMEM((_W, _NE, 3 * _EB), _F32),
        pltpu.VMEM((_W, _NE, _EB), _F32),
        pltpu.VMEM((_W, _NE, _EB), _F32),
        pltpu.VMEM((_W, _NE, _EB), _F32),
        pltpu.VMEM((_W, _NE, _EB), _F32),
    ]

    bfspec = pl.BlockSpec((_W, _NE, _NBF), lambda b: (b, 0, 0))
    bfshape = jax.ShapeDtypeStruct((_B, _NE, _NBF), _F32)
    jas, bx, by, bz = pl.pallas_call(
        _painn_kernel,
        grid=(_B // _W,),
        in_specs=in_specs,
        out_specs=[pl.BlockSpec((_W, 1), lambda b: (b, 0)),
                   bfspec, bfspec, bfspec],
        out_shape=[jax.ShapeDtypeStruct((_B, 1), _F32),
                   bfshape, bfshape, bfshape],
        scratch_shapes=scratch,
        compiler_params=pltpu.CompilerParams(
            dimension_semantics=("parallel",),
        ),
    )(*ops)

    jastrow = jas[:, 0]
    backflow = jnp.stack([bx, by, bz], axis=-1).transpose(0, 2, 1, 3)
    return jastrow, backflow


# speedup vs baseline: 1.2721x; 1.0682x over previous
import numpy as np
import jax
import jax.numpy as jnp
from jax.experimental import pallas as pl
from jax.experimental.pallas import tpu as pltpu

_B = 512
_NE = 18
_NN = 4
_NP = _NE * _NE
_NPN = _NN * _NE
_EB = 128
_NRBF = 20
_NF = _NRBF + 1
_NFP = 24
_CUT = 5.0
_L = 3
_NBF = 8
_W = 16
_F32 = jnp.float32
_BF16 = jnp.bfloat16


def _silu(x):
    return x * jax.nn.sigmoid(x)


def _mm(a, b):
    return jnp.dot(a, b, preferred_element_type=jnp.float32)


def _mmT(a, b):
    return jax.lax.dot_general(a, b, (((0,), (0,)), ((), ())),
                               preferred_element_type=jnp.float32)


def _update(s, vx, vy, vz, U, V, w1, b1, w2, b2):
    n = _W * _NE
    vcat = jnp.concatenate([vx, vy, vz], axis=0)
    Uv = _mm(vcat, U)
    Vv = _mm(vcat, V)
    Uvx, Uvy, Uvz = Uv[:n], Uv[n:2 * n], Uv[2 * n:]
    Vvx, Vvy, Vvz = Vv[:n], Vv[n:2 * n], Vv[2 * n:]
    Vn = jnp.sqrt(Vvx * Vvx + Vvy * Vvy + Vvz * Vvz + 1e-8)
    cat = jnp.concatenate([s, Vn], axis=-1)
    a = _mm(_silu(_mm(cat, w1) + b1), w2) + b2
    uvdot = Uvx * Vvx + Uvy * Vvy + Uvz * Vvz
    s = s + a[:, :_EB] + a[:, _EB:2 * _EB] * uvdot
    g = a[:, 2 * _EB:]
    return s, vx + g * Uvx, vy + g * Uvy, vz + g * Uvz


def _features(dx, dy, dz, mask):
    d = jnp.sqrt(dx * dx + dy * dy + dz * dz + 1e-12)
    dc = jnp.maximum(d, 1e-6)
    inv = 1.0 / dc
    th = d * (np.pi / _CUT)
    s1 = jnp.sin(th)
    c1 = jnp.cos(th)
    fc = jnp.where(d < _CUT, 0.5 * (c1 + 1.0), 0.0)
    if mask is not None:
        fc = fc * mask
    g = inv * fc
    base = []
    sq, cq = s1, c1
    base.append(sq * g)
    for _ in range(_NRBF - 1):
        sq, cq = sq * c1 + cq * s1, cq * c1 - sq * s1
        base.append(sq * g)
    base.append(fc)
    ux = dx * inv
    uy = dy * inv
    uz = dz * inv
    pad = [jnp.zeros_like(fc)] * (_NFP - _NF)
    return (base + pad + [r * ux for r in base] + pad
            + [r * uy for r in base] + pad
            + [r * uz for r in base] + pad)


def _painn_kernel(
    rsl_ref, rnt_ref, X_ref, Y_ref, TE_ref, TJ_ref, TN_ref, EN_ref, mask_ref,
    ee_w1, ee_b1, ee_w2, ee_b2, ee_wfa,
    ne_w1, ne_b1, ne_w2, ne_b2, ne_wfa,
    ue_U, ue_V, ue_w1, ue_b1, ue_w2, ue_b2,
    un_U, un_V, un_w1, un_b1, un_w2, un_b2,
    jw1, jb1, jw2, jb2, bfw,
    jas_ref, bx_ref, by_ref, bz_ref,
    fa_ref, fan_ref, phi_ref, pv_ref,
    ds_ref, dvx_ref, dvy_ref, dvz_ref,
):
    W = _W
    n = W * _NE

    rx = rsl_ref[:, 0, :]
    ry = rsl_ref[:, 1, :]
    rz = rsl_ref[:, 2, :]
    TE = TE_ref[:]
    TJ = TJ_ref[:]
    dx = _mm(rx, TE) - _mm(rx, TJ)
    dy = _mm(ry, TE) - _mm(ry, TJ)
    dz = _mm(rz, TE) - _mm(rz, TJ)
    rows = _features(dx, dy, dz, mask_ref[:])
    for q in range(4 * _NFP):
        fa_ref[:, q, :] = rows[q]

    TN = TN_ref[:]
    EN = EN_ref[:]
    rnt = rnt_ref[:]
    dxn = _mm(rx, TN) - _mm(rnt[0:1, :], EN)
    dyn = _mm(ry, TN) - _mm(rnt[1:2, :], EN)
    dzn = _mm(rz, TN) - _mm(rnt[2:3, :], EN)
    rowsn = _features(dxn, dyn, dzn, None)
    for q in range(4 * _NFP):
        fan_ref[:, q, :] = rowsn[q]

    Yv = Y_ref[:]

    s = jnp.broadcast_to(X_ref[:], (n, _EB))
    vx = jnp.zeros((n, _EB), _F32)
    vy = jnp.zeros((n, _EB), _F32)
    vz = jnp.zeros((n, _EB), _F32)

    for l in range(_L):
        h = _silu(_mm(s, ee_w1[l]) + ee_b1[l])
        phi = _mm(h, ee_w2[l]) + ee_b2[l]
        phi_ref[:] = phi.reshape(W, _NE, 3 * _EB)
        pvv = phi[:, _EB:2 * _EB]
        pv_ref[:] = jnp.concatenate(
            [pvv * vx, pvv * vy, pvv * vz], axis=-1).reshape(W, _NE, 3 * _EB)
        wfa = ee_wfa[l]
        wfvd = wfa[:, 2 * _EB:]

        def ee_body(b, carry):
            A = fa_ref[b]
            wfb = _mmT(A[:_NFP], wfa).reshape(_NE, _NE, 3 * _EB)
            wdx = _mmT(A[_NFP:2 * _NFP], wfvd).reshape(_NE, _NE, _EB)
            wdy = _mmT(A[2 * _NFP:3 * _NFP], wfvd).reshape(_NE, _NE, _EB)
            wdz = _mmT(A[3 * _NFP:], wfvd).reshape(_NE, _NE, _EB)
            phib = phi_ref[b]
            pvb = pv_ref[b]
            pd = phib[None, :, 2 * _EB:]
            ds_ref[b] = jnp.sum(phib[None, :, :_EB] * wfb[:, :, :_EB],
                                axis=1)
            wvv = wfb[:, :, _EB:2 * _EB]
            dvx_ref[b] = jnp.sum(pvb[None, :, :_EB] * wvv + pd * wdx, axis=1)
            dvy_ref[b] = jnp.sum(pvb[None, :, _EB:2 * _EB] * wvv + pd * wdy,
                                 axis=1)
            dvz_ref[b] = jnp.sum(pvb[None, :, 2 * _EB:] * wvv + pd * wdz,
                                 axis=1)
            return carry

        jax.lax.fori_loop(0, W, ee_body, 0)
        s = s + ds_ref[:].reshape(n, _EB)
        vx = vx + dvx_ref[:].reshape(n, _EB)
        vy = vy + dvy_ref[:].reshape(n, _EB)
        vz = vz + dvz_ref[:].reshape(n, _EB)

        s, vx, vy, vz = _update(s, vx, vy, vz, ue_U[l], ue_V[l], ue_w1[l],
                                ue_b1[l], ue_w2[l], ue_b2[l])

        hn = _silu(_mm(Yv, ne_w1[l]) + ne_b1[l])
        phin = _mm(hn, ne_w2[l]) + ne_b2[l]
        wfna = ne_wfa[l]
        wfnvd = wfna[:, 2 * _EB:]

        def ne_body(b, carry):
            A = fan_ref[b]
            wfb = _mmT(A[:_NFP], wfna).reshape(_NN, _NE, 3 * _EB)
            wdx = _mmT(A[_NFP:2 * _NFP], wfnvd).reshape(_NN, _NE, _EB)
            wdy = _mmT(A[2 * _NFP:3 * _NFP], wfnvd).reshape(_NN, _NE, _EB)
            wdz = _mmT(A[3 * _NFP:], wfnvd).reshape(_NN, _NE, _EB)
            pd = phin[:, None, 2 * _EB:]
            ds_ref[b] = jnp.sum(phin[:, None, :_EB] * wfb[:, :, :_EB],
                                axis=0)
            dvx_ref[b] = jnp.sum(pd * wdx, axis=0)
            dvy_ref[b] = jnp.sum(pd * wdy, axis=0)
            dvz_ref[b] = jnp.sum(pd * wdz, axis=0)
            return carry

        jax.lax.fori_loop(0, W, ne_body, 0)
        s = s + ds_ref[:].reshape(n, _EB)
        vx = vx + dvx_ref[:].reshape(n, _EB)
        vy = vy + dvy_ref[:].reshape(n, _EB)
        vz = vz + dvz_ref[:].reshape(n, _EB)

        s, vx, vy, vz = _update(s, vx, vy, vz, un_U[l], un_V[l], un_w1[l],
                                un_b1[l], un_w2[l], un_b2[l])

    hsum = jnp.sum(s.reshape(W, _NE, _EB), axis=1)
    jas_ref[:] = _mm(_silu(_mm(hsum, jw1[:]) + jb1[:]), jw2[:]) + jb2[:]
    bx_ref[:] = _mm(vx, bfw[:]).reshape(W, _NE, _NBF)
    by_ref[:] = _mm(vy, bfw[:]).reshape(W, _NE, _NBF)
    bz_ref[:] = _mm(vz, bfw[:]).reshape(W, _NE, _NBF)


@jax.jit
def kernel(rs, rn, params):
    p = params
    L = _L
    ee, ne = p['msg_ee'], p['msg_ne']
    ue, un = p['upd_e'], p['upd_n']

    zpad = jnp.zeros((L, _NFP - _NF, 3 * _EB), _F32)
    ee_wfa = jnp.concatenate(
        [ee['wf'], ee['bf'][:, None, :], zpad], axis=1)
    ne_wfa = jnp.concatenate(
        [ne['wf'], ne['bf'][:, None, :], zpad], axis=1)

    ii, jj = np.meshgrid(np.arange(_NE), np.arange(_NE), indexing='ij')
    TE = np.zeros((_NE, _NP), np.float32)
    TE[ii.reshape(-1), np.arange(_NP)] = 1.0
    TJ = np.zeros((_NE, _NP), np.float32)
    TJ[jj.reshape(-1), np.arange(_NP)] = 1.0
    mask = (ii != jj).reshape(1, _NP).astype(np.float32)
    aa, ei = np.meshgrid(np.arange(_NN), np.arange(_NE), indexing='ij')
    TN = np.zeros((_NE, _NPN), np.float32)
    TN[ei.reshape(-1), np.arange(_NPN)] = 1.0
    EN = np.zeros((_NN, _NPN), np.float32)
    EN[aa.reshape(-1), np.arange(_NPN)] = 1.0

    ops = [
        jnp.swapaxes(rs, 1, 2),
        rn.T,
        p['X'],
        p['Y'],
        jnp.asarray(TE), jnp.asarray(TJ), jnp.asarray(TN), jnp.asarray(EN),
        jnp.asarray(mask),
        ee['w1'], ee['b1'].reshape(L, 1, _EB),
        ee['w2'], ee['b2'].reshape(L, 1, 3 * _EB),
        ee_wfa,
        ne['w1'], ne['b1'].reshape(L, 1, _EB),
        ne['w2'], ne['b2'].reshape(L, 1, 3 * _EB),
        ne_wfa,
        ue['U'], ue['V'],
        ue['w1'], ue['b1'].reshape(L, 1, _EB),
        ue['w2'], ue['b2'].reshape(L, 1, 3 * _EB),
        un['U'], un['V'],
        un['w1'], un['b1'].reshape(L, 1, _EB),
        un['w2'], un['b2'].reshape(L, 1, 3 * _EB),
        p['jw1'], p['jb1'].reshape(1, _EB),
        p['jw2'], p['jb2'].reshape(1, 1),
        p['bf'],
    ]

    def rep(shape):
        nd = len(shape)
        return pl.BlockSpec(shape, lambda b, _nd=nd: (0,) * _nd)

    in_specs = [pl.BlockSpec((_W, 3, _NE), lambda b: (b, 0, 0))]
    in_specs += [rep(o.shape) for o in ops[1:]]

    scratch = [
        pltpu.VMEM((_W, 4 * _NFP, _NP), _F32),
        pltpu.VMEM((_W, 4 * _NFP, _NPN), _F32),
        pltpu.VMEM((_W, _NE, 3 * _EB), _F32),
        pltpu.VMEM((_W, _NE, 3 * _EB), _F32),
        pltpu.VMEM((_W, _NE, _EB), _F32),
        pltpu.VMEM((_W, _NE, _EB), _F32),
        pltpu.VMEM((_W, _NE, _EB), _F32),
        pltpu.VMEM((_W, _NE, _EB), _F32),
    ]

    bfspec = pl.BlockSpec((_W, _NE, _NBF), lambda b: (b, 0, 0))
    bfshape = jax.ShapeDtypeStruct((_B, _NE, _NBF), _F32)
    jas, bx, by, bz = pl.pallas_call(
        _painn_kernel,
        grid=(_B // _W,),
        in_specs=in_specs,
        out_specs=[pl.BlockSpec((_W, 1), lambda b: (b, 0)),
                   bfspec, bfspec, bfspec],
        out_shape=[jax.ShapeDtypeStruct((_B, 1), _F32),
                   bfshape, bfshape, bfshape],
        scratch_shapes=scratch,
        compiler_params=pltpu.CompilerParams(
            dimension_semantics=("parallel",),
        ),
    )(*ops)

    jastrow = jas[:, 0]
    backflow = jnp.stack([bx, by, bz], axis=-1).transpose(0, 2, 1, 3)
    return jastrow, backflow


# speedup vs baseline: 1.3166x; 1.0350x over previous
import numpy as np
import jax
import jax.numpy as jnp
from jax.experimental import pallas as pl
from jax.experimental.pallas import tpu as pltpu

_B = 512
_NE = 18
_NN = 4
_NP = _NE * _NE
_NPN = _NN * _NE
_EB = 128
_NRBF = 20
_NF = _NRBF + 1
_NFP = 24
_CUT = 5.0
_L = 3
_NBF = 8
_W = 32
_F32 = jnp.float32
_BF16 = jnp.bfloat16


def _silu(x):
    return x * jax.nn.sigmoid(x)


def _mm(a, b):
    return jnp.dot(a, b, preferred_element_type=jnp.float32)


def _mmT(a, b):
    return jax.lax.dot_general(a, b, (((0,), (0,)), ((), ())),
                               preferred_element_type=jnp.float32)


def _update(s, vx, vy, vz, U, V, w1, b1, w2, b2):
    n = _W * _NE
    vcat = jnp.concatenate([vx, vy, vz], axis=0)
    Uv = _mm(vcat, U)
    Vv = _mm(vcat, V)
    Uvx, Uvy, Uvz = Uv[:n], Uv[n:2 * n], Uv[2 * n:]
    Vvx, Vvy, Vvz = Vv[:n], Vv[n:2 * n], Vv[2 * n:]
    Vn = jnp.sqrt(Vvx * Vvx + Vvy * Vvy + Vvz * Vvz + 1e-8)
    cat = jnp.concatenate([s, Vn], axis=-1)
    a = _mm(_silu(_mm(cat, w1) + b1), w2) + b2
    uvdot = Uvx * Vvx + Uvy * Vvy + Uvz * Vvz
    s = s + a[:, :_EB] + a[:, _EB:2 * _EB] * uvdot
    g = a[:, 2 * _EB:]
    return s, vx + g * Uvx, vy + g * Uvy, vz + g * Uvz


def _features(dx, dy, dz, mask):
    d = jnp.sqrt(dx * dx + dy * dy + dz * dz + 1e-12)
    dc = jnp.maximum(d, 1e-6)
    inv = 1.0 / dc
    th = d * (np.pi / _CUT)
    s1 = jnp.sin(th)
    c1 = jnp.cos(th)
    fc = jnp.where(d < _CUT, 0.5 * (c1 + 1.0), 0.0)
    if mask is not None:
        fc = fc * mask
    g = inv * fc
    base = []
    sq, cq = s1, c1
    base.append(sq * g)
    for _ in range(_NRBF - 1):
        sq, cq = sq * c1 + cq * s1, cq * c1 - sq * s1
        base.append(sq * g)
    base.append(fc)
    ux = dx * inv
    uy = dy * inv
    uz = dz * inv
    pad = [jnp.zeros_like(fc)] * (_NFP - _NF)
    return (base + pad + [r * ux for r in base] + pad
            + [r * uy for r in base] + pad
            + [r * uz for r in base] + pad)


def _painn_kernel(
    rsl_ref, rnt_ref, X_ref, Y_ref, TE_ref, TJ_ref, TN_ref, EN_ref, mask_ref,
    ee_w1, ee_b1, ee_w2, ee_b2, ee_wfa,
    ne_w1, ne_b1, ne_w2, ne_b2, ne_wfa,
    ue_U, ue_V, ue_w1, ue_b1, ue_w2, ue_b2,
    un_U, un_V, un_w1, un_b1, un_w2, un_b2,
    jw1, jb1, jw2, jb2, bfw,
    jas_ref, bx_ref, by_ref, bz_ref,
    fa_ref, fan_ref, phi_ref, pv_ref,
    ds_ref, dvx_ref, dvy_ref, dvz_ref,
):
    W = _W
    n = W * _NE

    rx = rsl_ref[:, 0, :]
    ry = rsl_ref[:, 1, :]
    rz = rsl_ref[:, 2, :]
    TE = TE_ref[:]
    TJ = TJ_ref[:]
    dx = _mm(rx, TE) - _mm(rx, TJ)
    dy = _mm(ry, TE) - _mm(ry, TJ)
    dz = _mm(rz, TE) - _mm(rz, TJ)
    rows = _features(dx, dy, dz, mask_ref[:])
    for q in range(4 * _NFP):
        fa_ref[:, q, :] = rows[q]

    TN = TN_ref[:]
    EN = EN_ref[:]
    rnt = rnt_ref[:]
    dxn = _mm(rx, TN) - _mm(rnt[0:1, :], EN)
    dyn = _mm(ry, TN) - _mm(rnt[1:2, :], EN)
    dzn = _mm(rz, TN) - _mm(rnt[2:3, :], EN)
    rowsn = _features(dxn, dyn, dzn, None)
    for q in range(4 * _NFP):
        fan_ref[:, q, :] = rowsn[q]

    Yv = Y_ref[:]

    s = jnp.broadcast_to(X_ref[:], (n, _EB))
    vx = jnp.zeros((n, _EB), _F32)
    vy = jnp.zeros((n, _EB), _F32)
    vz = jnp.zeros((n, _EB), _F32)

    for l in range(_L):
        h = _silu(_mm(s, ee_w1[l]) + ee_b1[l])
        phi = _mm(h, ee_w2[l]) + ee_b2[l]
        phi_ref[:] = phi.reshape(W, _NE, 3 * _EB)
        pvv = phi[:, _EB:2 * _EB]
        pv_ref[:] = jnp.concatenate(
            [pvv * vx, pvv * vy, pvv * vz], axis=-1).reshape(W, _NE, 3 * _EB)
        wfa = ee_wfa[l]
        wfvd = wfa[:, 2 * _EB:]

        def ee_body(b, carry):
            A = fa_ref[b]
            wfb = _mmT(A[:_NFP], wfa).reshape(_NE, _NE, 3 * _EB)
            wdx = _mmT(A[_NFP:2 * _NFP], wfvd).reshape(_NE, _NE, _EB)
            wdy = _mmT(A[2 * _NFP:3 * _NFP], wfvd).reshape(_NE, _NE, _EB)
            wdz = _mmT(A[3 * _NFP:], wfvd).reshape(_NE, _NE, _EB)
            phib = phi_ref[b]
            pvb = pv_ref[b]
            pd = phib[None, :, 2 * _EB:]
            ds_ref[b] = jnp.sum(phib[None, :, :_EB] * wfb[:, :, :_EB],
                                axis=1)
            wvv = wfb[:, :, _EB:2 * _EB]
            dvx_ref[b] = jnp.sum(pvb[None, :, :_EB] * wvv + pd * wdx, axis=1)
            dvy_ref[b] = jnp.sum(pvb[None, :, _EB:2 * _EB] * wvv + pd * wdy,
                                 axis=1)
            dvz_ref[b] = jnp.sum(pvb[None, :, 2 * _EB:] * wvv + pd * wdz,
                                 axis=1)
            return carry

        jax.lax.fori_loop(0, W, ee_body, 0)
        s = s + ds_ref[:].reshape(n, _EB)
        vx = vx + dvx_ref[:].reshape(n, _EB)
        vy = vy + dvy_ref[:].reshape(n, _EB)
        vz = vz + dvz_ref[:].reshape(n, _EB)

        s, vx, vy, vz = _update(s, vx, vy, vz, ue_U[l], ue_V[l], ue_w1[l],
                                ue_b1[l], ue_w2[l], ue_b2[l])

        hn = _silu(_mm(Yv, ne_w1[l]) + ne_b1[l])
        phin = _mm(hn, ne_w2[l]) + ne_b2[l]
        wfna = ne_wfa[l]
        wfnvd = wfna[:, 2 * _EB:]

        def ne_body(b, carry):
            A = fan_ref[b]
            wfb = _mmT(A[:_NFP], wfna).reshape(_NN, _NE, 3 * _EB)
            wdx = _mmT(A[_NFP:2 * _NFP], wfnvd).reshape(_NN, _NE, _EB)
            wdy = _mmT(A[2 * _NFP:3 * _NFP], wfnvd).reshape(_NN, _NE, _EB)
            wdz = _mmT(A[3 * _NFP:], wfnvd).reshape(_NN, _NE, _EB)
            pd = phin[:, None, 2 * _EB:]
            ds_ref[b] = jnp.sum(phin[:, None, :_EB] * wfb[:, :, :_EB],
                                axis=0)
            dvx_ref[b] = jnp.sum(pd * wdx, axis=0)
            dvy_ref[b] = jnp.sum(pd * wdy, axis=0)
            dvz_ref[b] = jnp.sum(pd * wdz, axis=0)
            return carry

        jax.lax.fori_loop(0, W, ne_body, 0)
        s = s + ds_ref[:].reshape(n, _EB)
        vx = vx + dvx_ref[:].reshape(n, _EB)
        vy = vy + dvy_ref[:].reshape(n, _EB)
        vz = vz + dvz_ref[:].reshape(n, _EB)

        s, vx, vy, vz = _update(s, vx, vy, vz, un_U[l], un_V[l], un_w1[l],
                                un_b1[l], un_w2[l], un_b2[l])

    hsum = jnp.sum(s.reshape(W, _NE, _EB), axis=1)
    jas_ref[:] = _mm(_silu(_mm(hsum, jw1[:]) + jb1[:]), jw2[:]) + jb2[:]
    bx_ref[:] = _mm(vx, bfw[:]).reshape(W, _NE, _NBF)
    by_ref[:] = _mm(vy, bfw[:]).reshape(W, _NE, _NBF)
    bz_ref[:] = _mm(vz, bfw[:]).reshape(W, _NE, _NBF)


@jax.jit
def kernel(rs, rn, params):
    p = params
    L = _L
    ee, ne = p['msg_ee'], p['msg_ne']
    ue, un = p['upd_e'], p['upd_n']

    zpad = jnp.zeros((L, _NFP - _NF, 3 * _EB), _F32)
    ee_wfa = jnp.concatenate(
        [ee['wf'], ee['bf'][:, None, :], zpad], axis=1)
    ne_wfa = jnp.concatenate(
        [ne['wf'], ne['bf'][:, None, :], zpad], axis=1)

    ii, jj = np.meshgrid(np.arange(_NE), np.arange(_NE), indexing='ij')
    TE = np.zeros((_NE, _NP), np.float32)
    TE[ii.reshape(-1), np.arange(_NP)] = 1.0
    TJ = np.zeros((_NE, _NP), np.float32)
    TJ[jj.reshape(-1), np.arange(_NP)] = 1.0
    mask = (ii != jj).reshape(1, _NP).astype(np.float32)
    aa, ei = np.meshgrid(np.arange(_NN), np.arange(_NE), indexing='ij')
    TN = np.zeros((_NE, _NPN), np.float32)
    TN[ei.reshape(-1), np.arange(_NPN)] = 1.0
    EN = np.zeros((_NN, _NPN), np.float32)
    EN[aa.reshape(-1), np.arange(_NPN)] = 1.0

    ops = [
        jnp.swapaxes(rs, 1, 2),
        rn.T,
        p['X'],
        p['Y'],
        jnp.asarray(TE), jnp.asarray(TJ), jnp.asarray(TN), jnp.asarray(EN),
        jnp.asarray(mask),
        ee['w1'], ee['b1'].reshape(L, 1, _EB),
        ee['w2'], ee['b2'].reshape(L, 1, 3 * _EB),
        ee_wfa,
        ne['w1'], ne['b1'].reshape(L, 1, _EB),
        ne['w2'], ne['b2'].reshape(L, 1, 3 * _EB),
        ne_wfa,
        ue['U'], ue['V'],
        ue['w1'], ue['b1'].reshape(L, 1, _EB),
        ue['w2'], ue['b2'].reshape(L, 1, 3 * _EB),
        un['U'], un['V'],
        un['w1'], un['b1'].reshape(L, 1, _EB),
        un['w2'], un['b2'].reshape(L, 1, 3 * _EB),
        p['jw1'], p['jb1'].reshape(1, _EB),
        p['jw2'], p['jb2'].reshape(1, 1),
        p['bf'],
    ]

    def rep(shape):
        nd = len(shape)
        return pl.BlockSpec(shape, lambda b, _nd=nd: (0,) * _nd)

    in_specs = [pl.BlockSpec((_W, 3, _NE), lambda b: (b, 0, 0))]
    in_specs += [rep(o.shape) for o in ops[1:]]

    scratch = [
        pltpu.VMEM((_W, 4 * _NFP, _NP), _F32),
        pltpu.VMEM((_W, 4 * _NFP, _NPN), _F32),
        pltpu.VMEM((_W, _NE, 3 * _EB), _F32),
        pltpu.VMEM((_W, _NE, 3 * _EB), _F32),
        pltpu.VMEM((_W, _NE, _EB), _F32),
        pltpu.VMEM((_W, _NE, _EB), _F32),
        pltpu.VMEM((_W, _NE, _EB), _F32),
        pltpu.VMEM((_W, _NE, _EB), _F32),
    ]

    bfspec = pl.BlockSpec((_W, _NE, _NBF), lambda b: (b, 0, 0))
    bfshape = jax.ShapeDtypeStruct((_B, _NE, _NBF), _F32)
    jas, bx, by, bz = pl.pallas_call(
        _painn_kernel,
        grid=(_B // _W,),
        in_specs=in_specs,
        out_specs=[pl.BlockSpec((_W, 1), lambda b: (b, 0)),
                   bfspec, bfspec, bfspec],
        out_shape=[jax.ShapeDtypeStruct((_B, 1), _F32),
                   bfshape, bfshape, bfshape],
        scratch_shapes=scratch,
        compiler_params=pltpu.CompilerParams(
            dimension_semantics=("parallel",),
        ),
    )(*ops)

    jastrow = jas[:, 0]
    backflow = jnp.stack([bx, by, bz], axis=-1).transpose(0, 2, 1, 3)
    return jastrow, backflow
